# baseline (speedup 1.0000x reference)
import jax
import jax.numpy as jnp
from jax import lax
from jax.experimental import pallas as pl
from jax.experimental.pallas import tpu as pltpu
from jax.experimental.pallas import tpu_sc as plsc

B = 16384
D = 32
NC = 2
NS = 16
NW = NC * NS
BPW = B // NW
L = 16
GROUPS = BPW // L


def _mf_dot_body(uid_hbm, iid_hbm, ut_hbm, it_hbm, out_hbm,
                 uidx_v, iidx_v, urows_v, irows_v, outc_v, ptmp_v, usem, isem):
    wid = lax.axis_index("s") * NC + lax.axis_index("c")
    base = wid * BPW
    pltpu.sync_copy(uid_hbm.at[pl.ds(base, BPW)], uidx_v)
    pltpu.sync_copy(iid_hbm.at[pl.ds(base, BPW)], iidx_v)
    cu = pltpu.async_copy(ut_hbm.at[uidx_v], urows_v, usem)
    ci = pltpu.async_copy(it_hbm.at[iidx_v], irows_v, isem)
    cu.wait()
    ci.wait()

    lanes = lax.iota(jnp.int32, L)
    perms = [jnp.bitwise_xor(lanes, sh) for sh in (8, 4, 2, 1)]
    gd = lax.GatherDimensionNumbers(
        offset_dims=(), collapsed_slice_dims=(0,), start_index_map=(0,))

    def shuf(x, idx):
        return lax.gather(x, idx[:, None], gd, slice_sizes=(1,),
                          mode=lax.GatherScatterMode.PROMISE_IN_BOUNDS)

    def group(g, carry):
        base_r = g * L
        acc = jnp.zeros((L,), jnp.float32)
        for j in range(L):
            r = base_r + j
            u0 = urows_v[r, pl.ds(0, L)]
            u1 = urows_v[r, pl.ds(L, L)]
            v0 = irows_v[r, pl.ds(0, L)]
            v1 = irows_v[r, pl.ds(L, L)]
            p = u0 * v0 + u1 * v1
            for perm in perms:
                p = p + shuf(p, perm)
            acc = jnp.where(lanes == j, p, acc)
        outc_v[pl.ds(base_r, L)] = acc
        return carry

    lax.fori_loop(0, GROUPS, group, 0)
    pltpu.sync_copy(outc_v, out_hbm.at[pl.ds(base, BPW)])


def kernel(user_ids, item_ids, user_table, item_table):
    mesh = plsc.VectorSubcoreMesh(core_axis_name="c", subcore_axis_name="s")
    k = pl.kernel(
        _mf_dot_body,
        mesh=mesh,
        compiler_params=pltpu.CompilerParams(use_tc_tiling_on_sc=False),
        out_type=jax.ShapeDtypeStruct((B,), jnp.float32),
        scratch_types=[
            pltpu.VMEM((BPW,), jnp.int32),
            pltpu.VMEM((BPW,), jnp.int32),
            pltpu.VMEM((BPW, D), jnp.float32),
            pltpu.VMEM((BPW, D), jnp.float32),
            pltpu.VMEM((BPW,), jnp.float32),
            pltpu.VMEM((L * L,), jnp.float32),
            pltpu.SemaphoreType.DMA,
            pltpu.SemaphoreType.DMA,
        ],
    )
    return k(user_ids, item_ids, user_table, item_table)


# speedup vs baseline: 4.1492x; 4.1492x over previous
import jax
import jax.numpy as jnp
from jax import lax
from jax.experimental import pallas as pl
from jax.experimental.pallas import tpu as pltpu
from jax.experimental.pallas import tpu_sc as plsc

B = 16384
D = 32
NC = 2
NS = 16
NW = NC * NS
BPW = B // NW
L = 16
DEPTH = 8
NSG = BPW // DEPTH
TCOLS = 1000000


def _mf_dot_body(uid_hbm, iid_hbm, ut_hbm, it_hbm, out_hbm,
                 uidx_v, iidx_v, outc_v, rects, usem, isem, osem):
    urects = rects[:DEPTH]
    irects = rects[DEPTH:]
    wid = lax.axis_index("s") * NC + lax.axis_index("c")
    base = wid * BPW
    pltpu.sync_copy(uid_hbm.at[pl.ds(base, BPW)], uidx_v.at[pl.ds(0, BPW)])
    pltpu.sync_copy(iid_hbm.at[pl.ds(base, BPW)], iidx_v.at[pl.ds(0, BPW)])

    lanes = lax.iota(jnp.int32, L)
    rows_lo = lanes
    rows_hi = lanes + L
    perms = [jnp.bitwise_xor(lanes, sh) for sh in (8, 4, 2, 1)]
    gd = lax.GatherDimensionNumbers(
        offset_dims=(), collapsed_slice_dims=(0,), start_index_map=(0,))

    def shuf(x, idx):
        return lax.gather(x, idx[:, None], gd, slice_sizes=(1,),
                          mode=lax.GatherScatterMode.PROMISE_IN_BOUNDS)

    def col_off(idx_scalar):
        return pl.multiple_of((idx_scalar // 128) * 128, 128)

    def issue(j, u, i):
        pltpu.async_copy(ut_hbm.at[:, pl.ds(col_off(u), 128)], urects[j], usem)
        pltpu.async_copy(it_hbm.at[:, pl.ds(col_off(i), 128)], irects[j], isem)

    uc0 = uidx_v[pl.ds(0, L)]
    ic0 = iidx_v[pl.ds(0, L)]
    for j in range(DEPTH):
        issue(j, uc0[j], ic0[j])

    def supergroup(sg, carry):
        uc = uidx_v[pl.ds(sg * DEPTH, L)]
        ic = iidx_v[pl.ds(sg * DEPTH, L)]
        ucn = uidx_v[pl.ds(sg * DEPTH + DEPTH, L)]
        icn = iidx_v[pl.ds(sg * DEPTH + DEPTH, L)]
        in_range = sg < NSG - 1
        w = jnp.zeros((L,), jnp.float32)
        for j in range(DEPTH):
            pltpu.make_async_copy(
                ut_hbm.at[:, pl.ds(0, 128)], urects[j], usem).wait()
            pltpu.make_async_copy(
                it_hbm.at[:, pl.ds(0, 128)], irects[j], isem).wait()
            u = uc[j]
            i = ic[j]
            ul = jnp.full((L,), u % 128, jnp.int32)
            il = jnp.full((L,), i % 128, jnp.int32)
            e0 = plsc.load_gather(urects[j], [rows_lo, ul])
            e1 = plsc.load_gather(urects[j], [rows_hi, ul])
            f0 = plsc.load_gather(irects[j], [rows_lo, il])
            f1 = plsc.load_gather(irects[j], [rows_hi, il])
            p = e0 * f0 + e1 * f1
            for perm in perms:
                p = p + shuf(p, perm)
            w = jnp.where(lanes == j, p, w)
            un = jnp.where(in_range, ucn[j], 0)
            vn = jnp.where(in_range, icn[j], 0)
            issue(j, un, vn)
        outc_v[pl.ds(sg * DEPTH, L)] = w
        return carry

    lax.fori_loop(0, NSG, supergroup, 0)
    for j in range(DEPTH):
        pltpu.make_async_copy(
            ut_hbm.at[:, pl.ds(0, 128)], urects[j], usem).wait()
        pltpu.make_async_copy(
            it_hbm.at[:, pl.ds(0, 128)], irects[j], isem).wait()
    pltpu.async_copy(
        outc_v.at[pl.ds(0, BPW)], out_hbm.at[pl.ds(base, BPW)], osem).wait()


def kernel(user_ids, item_ids, user_table, item_table):
    mesh = plsc.VectorSubcoreMesh(core_axis_name="c", subcore_axis_name="s")
    k = pl.kernel(
        _mf_dot_body,
        mesh=mesh,
        compiler_params=pltpu.CompilerParams(needs_layout_passes=False),
        out_type=jax.ShapeDtypeStruct((B,), jnp.float32),
        scratch_types=[
            pltpu.VMEM((BPW + 2 * L,), jnp.int32),
            pltpu.VMEM((BPW + 2 * L,), jnp.int32),
            pltpu.VMEM((BPW + 2 * L,), jnp.float32),
            [pltpu.VMEM((D, 128), jnp.float32) for _ in range(2 * DEPTH)],
            pltpu.SemaphoreType.DMA,
            pltpu.SemaphoreType.DMA,
            pltpu.SemaphoreType.DMA,
        ],
    )
    return k(user_ids, item_ids, user_table.T, item_table.T)
